# Initial kernel scaffold; baseline (speedup 1.0000x reference)
import jax
import jax.numpy as jnp
from jax.experimental import pallas as pl


def kernel(x, edge_index, edge_weights, W1, b1, W2, b2, fc1_W, fc1_b, fc2_W, fc2_b):
    raise NotImplementedError("write your pallas kernel here")


# speedup vs baseline: 33.5950x; 33.5950x over previous
import dataclasses
import functools

import jax
import jax.numpy as jnp
from jax import lax
from jax.experimental import pallas as pl
from jax.experimental.pallas import tpu as pltpu
from jax.experimental.pallas import tpu_sc as plsc

_F32 = jnp.float32
_NC = 2
_NS = 16
_NW = _NC * _NS
_R = 80
_CB = 25


def _sc_mesh():
    return plsc.VectorSubcoreMesh(core_axis_name="c", subcore_axis_name="s")


def _sc_params():
    cp = pltpu.CompilerParams()
    fields = pltpu.CompilerParams.__dataclass_fields__
    if "needs_layout_passes" in fields:
        cp = dataclasses.replace(cp, needs_layout_passes=False)
    if "use_tc_tiling_on_sc" in fields:
        cp = dataclasses.replace(cp, use_tc_tiling_on_sc=False)
    return cp


@functools.cache
def _deg_kernel(n_nodes, nb):
    def body(dst_hbm, ew_hbm, out_hbm, dstv, ewv, degloc):
        cid = lax.axis_index("c")
        sid = lax.axis_index("s")
        wid = sid * _NC + cid
        z16 = jnp.zeros((16,), _F32)
        zi16 = jnp.zeros((16,), jnp.int32)

        @pl.loop(0, n_nodes, step=16)
        def _(i):
            degloc[0, pl.ds(i, 16)] = z16

        @pl.loop(0, nb)
        def _(b):
            pltpu.sync_copy(dst_hbm.at[wid, b], dstv)
            pltpu.sync_copy(ew_hbm.at[wid, b], ewv)

            @pl.loop(0, _CB)
            def _(j):
                @pl.loop(0, _R, step=16)
                def _(k):
                    idx = dstv[j, pl.ds(k, 16)]
                    vals = ewv[j, pl.ds(k, 16)]
                    plsc.addupdate_scatter(degloc, [zi16, idx], vals)

        pltpu.sync_copy(degloc, out_hbm.at[wid])

    return pl.kernel(
        body,
        out_type=jax.ShapeDtypeStruct((_NW, 1, n_nodes), _F32),
        mesh=_sc_mesh(),
        compiler_params=_sc_params(),
        scratch_types=[
            pltpu.VMEM((_CB, _R), jnp.int32),
            pltpu.VMEM((_CB, _R), _F32),
            pltpu.VMEM((1, n_nodes), _F32),
        ],
    )


@functools.cache
def _agg_kernel(n_nodes, nb, dfeat):
    zr = 80
    nzb = n_nodes // zr
    nkv = dfeat // 16

    def body(g_hbm, src_hbm, dst_hbm, ew_hbm, out_hbm,
             srcv, dstv, ewv, rows, zbuf, acc, sem_g, sem_s):
        cid = lax.axis_index("c")
        sid = lax.axis_index("s")
        wid = sid * _NC + cid
        z16 = jnp.zeros((16,), _F32)

        @pl.loop(0, zr)
        def _(r2):
            for k in range(nkv):
                zbuf[r2, pl.ds(k * 16, 16)] = z16

        @pl.loop(sid, nzb, step=_NS)
        def _(zb):
            pltpu.sync_copy(zbuf, acc.at[pl.ds(zb * zr, zr)])

        plsc.subcore_barrier()

        @pl.loop(0, nb)
        def _(b):
            pltpu.sync_copy(src_hbm.at[wid, b], srcv)
            pltpu.sync_copy(dst_hbm.at[wid, b], dstv)
            pltpu.sync_copy(ew_hbm.at[wid, b], ewv)

            @pl.loop(0, _CB)
            def _(j):
                pltpu.async_copy(g_hbm.at[srcv.at[j]],
                                 rows.at[pl.ds(j * _R, _R)], sem_g)

            @pl.loop(0, _CB)
            def _(j):
                pltpu.make_async_copy(g_hbm.at[srcv.at[j]],
                                      rows.at[pl.ds(j * _R, _R)], sem_g).wait()
                jsplat = jnp.full((16,), j, jnp.int32)

                @pl.loop(0, _R)
                def _(e):
                    esplat = jnp.full((16,), e, jnp.int32)
                    ews = plsc.load_gather(ewv, [jsplat, esplat])
                    row = j * _R + e
                    for k in range(nkv):
                        sl = pl.ds(k * 16, 16)
                        rows[row, sl] = rows[row, sl] * ews

                pltpu.async_copy(rows.at[pl.ds(j * _R, _R)],
                                 acc.at[dstv.at[j]], sem_s, add=True)

            @pl.loop(0, _CB)
            def _(j):
                pltpu.make_async_copy(rows.at[pl.ds(j * _R, _R)],
                                      acc.at[dstv.at[j]], sem_s).wait()

        plsc.subcore_barrier()

        @pl.when(sid == 0)
        def _():
            pltpu.sync_copy(acc, out_hbm.at[cid])

    return pl.kernel(
        body,
        out_type=jax.ShapeDtypeStruct((_NC, n_nodes, dfeat), _F32),
        mesh=_sc_mesh(),
        compiler_params=_sc_params(),
        scratch_types=[
            pltpu.VMEM((_CB, _R), jnp.int32),
            pltpu.VMEM((_CB, _R), jnp.int32),
            pltpu.VMEM((_CB, _R), _F32),
            pltpu.VMEM((_CB * _R, dfeat), _F32),
            pltpu.VMEM((zr, dfeat), _F32),
            pltpu.VMEM_SHARED((n_nodes, dfeat), _F32),
            pltpu.SemaphoreType.DMA,
            pltpu.SemaphoreType.DMA,
        ],
    )


def _dot(a, b):
    return jax.lax.dot_general(a, b, (((1,), (0,)), ((), ())),
                               precision=jax.lax.Precision.HIGHEST,
                               preferred_element_type=_F32)


def _prep_a_body(dp_ref, dinv_ref, ideg_ref):
    deg = jnp.sum(dp_ref[...], axis=0) + 1.0
    pos = deg > 0
    dinv_ref[...] = jnp.where(pos, lax.rsqrt(deg), 0.0)
    ideg_ref[...] = jnp.where(pos, 1.0 / deg, 0.0)


def _prep_b_body(x_ref, w1_ref, dinv_ref, h1_ref, g1_ref):
    h1 = _dot(x_ref[...], w1_ref[...])
    h1_ref[...] = h1
    g1_ref[...] = h1 * dinv_ref[...]


def _mid_body(p1_ref, h1_ref, dinv_ref, ideg_ref, b1_ref, w2_ref,
              h_ref, h2_ref, g2_ref):
    s1 = p1_ref[0] + p1_ref[1]
    x1 = dinv_ref[...] * s1 + ideg_ref[...] * h1_ref[...] + b1_ref[...]
    h = jnp.maximum(x1, 0.0)
    h_ref[...] = h
    h2 = _dot(h, w2_ref[...])
    h2_ref[...] = h2
    g2_ref[...] = h2 * dinv_ref[...]


def _final_body(p2_ref, h2_ref, h_ref, dinv_ref, ideg_ref, b2_ref,
                fc1_wt_ref, fc1_b_ref, w_h_ref, w_c_ref, fc2_b_ref,
                r_ref, x1_out_ref):
    s2 = p2_ref[0] + p2_ref[1]
    x2 = dinv_ref[...] * s2 + ideg_ref[...] * h2_ref[...] + b2_ref[...]
    x1_out = _dot(x2, fc1_wt_ref[...]) + fc1_b_ref[...]
    x1_out_ref[...] = x1_out
    c = jax.nn.sigmoid(x1_out)
    r_ref[...] = _dot(h_ref[...], w_h_ref[...]) + c * w_c_ref[...] + fc2_b_ref[...]


def _tc_call(body, out_shapes):
    return pl.pallas_call(
        body,
        out_shape=[jax.ShapeDtypeStruct(s, _F32) for s in out_shapes])


_BN = 2000


def _row_spec(shape3=None, shape2=None):
    if shape3 is not None:
        return pl.BlockSpec((shape3[0], _BN, shape3[2]), lambda i: (0, i, 0))
    return pl.BlockSpec((_BN, shape2[1]), lambda i: (i, 0))


def _full_spec(shape):
    return pl.BlockSpec(shape, lambda i: tuple(0 for _ in shape))


def _tc_call_rows(body, n, out_minor, in_specs):
    grid = n // _BN
    return pl.pallas_call(
        body,
        grid=(grid,),
        in_specs=in_specs,
        out_specs=[_row_spec(shape2=(n, m)) for m in out_minor],
        out_shape=[jax.ShapeDtypeStruct((n, m), _F32) for m in out_minor])


def kernel(x, edge_index, edge_weights, W1, b1, W2, b2, fc1_W, fc1_b, fc2_W, fc2_b):
    n, d_in = x.shape
    e = edge_index.shape[1]
    d1 = W1.shape[1]
    d2 = W2.shape[1]
    epw = e // _NW
    nb = epw // (_CB * _R)

    eshape = (_NW, nb, _CB, _R)
    src4d = edge_index[0].reshape(eshape)
    dst4d = edge_index[1].reshape(eshape)
    ew4d = edge_weights.reshape(eshape)

    degpart = _deg_kernel(n, nb)(dst4d, ew4d).reshape(_NW, n)

    dinv1d, ideg1d = _tc_call(_prep_a_body, [(n,), (n,)])(degpart)
    dinv = dinv1d.reshape(n, 1)
    ideg = ideg1d.reshape(n, 1)

    h1, g1 = _tc_call(_prep_b_body, [(n, d1), (n, d1)])(x, W1, dinv)

    p1 = _agg_kernel(n, nb, d1)(g1, src4d, dst4d, ew4d)

    h, h2, g2 = _tc_call_rows(
        _mid_body, n, [d1, d2, d2],
        [_row_spec(shape3=(_NC, n, d1)), _row_spec(shape2=(n, d1)),
         _row_spec(shape2=(n, 1)), _row_spec(shape2=(n, 1)),
         _full_spec((1, d1)), _full_spec((d1, d2))])(
        p1, h1, dinv, ideg, b1.reshape(1, d1), W2)

    p2 = _agg_kernel(n, nb, d2)(g2, src4d, dst4d, ew4d)

    r, x1_out = _tc_call_rows(
        _final_body, n, [1, 1],
        [_row_spec(shape3=(_NC, n, d2)), _row_spec(shape2=(n, d2)),
         _row_spec(shape2=(n, d1)), _row_spec(shape2=(n, 1)),
         _row_spec(shape2=(n, 1)), _full_spec((1, d2)),
         _full_spec((d2, 1)), _full_spec((1, 1)),
         _full_spec((d1, 1)), _full_spec((1, 1)), _full_spec((1, 1))])(
        p2, h2, h, dinv, ideg, b2.reshape(1, d2),
        fc1_W.reshape(d2, 1), fc1_b.reshape(1, 1),
        fc2_W[:, :d1].reshape(d1, 1), fc2_W[:, d1:], fc2_b.reshape(1, 1))

    return (r, x1_out)


# speedup vs baseline: 46.6209x; 1.3877x over previous
import dataclasses
import functools

import jax
import jax.numpy as jnp
from jax import lax
from jax.experimental import pallas as pl
from jax.experimental.pallas import tpu as pltpu
from jax.experimental.pallas import tpu_sc as plsc

_F32 = jnp.float32
_I32 = jnp.int32
_NC = 2
_NS = 16
_NW = _NC * _NS
_R = 80
_CB = 25


def _sc_mesh():
    return plsc.VectorSubcoreMesh(core_axis_name="c", subcore_axis_name="s")


def _sc_params():
    cp = pltpu.CompilerParams()
    fields = pltpu.CompilerParams.__dataclass_fields__
    if "needs_layout_passes" in fields:
        cp = dataclasses.replace(cp, needs_layout_passes=False)
    if "use_tc_tiling_on_sc" in fields:
        cp = dataclasses.replace(cp, use_tc_tiling_on_sc=False)
    return cp


def _rsqrt_newton(d):
    i = plsc.bitcast(d, _I32)
    i = jnp.full((16,), 0x5F3759DF, _I32) - lax.shift_right_logical(
        i, jnp.full((16,), 1, _I32))
    y = plsc.bitcast(i, _F32)
    half = d * (-0.5)
    for _ in range(4):
        y = y * (half * y * y + 1.5)
    return y


@functools.cache
def _deg_dinv_kernel(n_nodes, nb):
    npart = nb * _CB
    nblk = n_nodes // _R

    def body(dst_hbm, ew_hbm, out_hbm, dstv, ewv, degloc, rbuf, dbuf,
             shdeg, sem):
        cid = lax.axis_index("c")
        sid = lax.axis_index("s")
        wid = sid * _NC + cid
        z16 = jnp.zeros((16,), _F32)
        zi16 = jnp.zeros((16,), _I32)

        @pl.loop(0, n_nodes, step=16)
        def _(i):
            degloc[0, pl.ds(i, 16)] = z16

        for half in range(2):
            p = sid * 2 + half
            cp_d = pltpu.async_copy(dst_hbm.at[p], dstv.at[half], sem)
            cp_w = pltpu.async_copy(ew_hbm.at[p], ewv.at[half], sem)
            cp_d.wait()
            cp_w.wait()

            @pl.loop(0, npart)
            def _(j):
                @pl.loop(0, _R, step=16)
                def _(k):
                    idx = dstv[half, j, pl.ds(k, 16)]
                    vals = ewv[half, j, pl.ds(k, 16)]
                    plsc.addupdate_scatter(degloc, [zi16, idx], vals)

        pltpu.sync_copy(degloc, shdeg.at[sid])
        plsc.subcore_barrier()

        @pl.loop(wid, nblk, step=_NW)
        def _(blk):
            pltpu.sync_copy(shdeg.at[:, 0, pl.ds(blk * _R, _R)], rbuf)

            @pl.loop(0, _R, step=16)
            def _(k):
                acc = rbuf[0, pl.ds(k, 16)] + 1.0
                for t in range(1, _NS):
                    acc = acc + rbuf[t, pl.ds(k, 16)]
                dbuf[0, pl.ds(k, 16)] = _rsqrt_newton(acc)

            pltpu.sync_copy(dbuf, out_hbm.at[:, pl.ds(blk * _R, _R)])

    return pl.kernel(
        body,
        out_type=jax.ShapeDtypeStruct((1, n_nodes), _F32),
        mesh=_sc_mesh(),
        compiler_params=_sc_params(),
        scratch_types=[
            pltpu.VMEM((2, npart, _R), _I32),
            pltpu.VMEM((2, npart, _R), _F32),
            pltpu.VMEM((1, n_nodes), _F32),
            pltpu.VMEM((_NS, _R), _F32),
            pltpu.VMEM((1, _R), _F32),
            pltpu.VMEM_SHARED((_NS, 1, n_nodes), _F32),
            pltpu.SemaphoreType.DMA,
        ],
    )


@functools.cache
def _agg_kernel(n_nodes, nb, dfeat):
    zr = 80
    nzb = n_nodes // zr
    nkv = dfeat // 16

    def body(g_hbm, dinv_hbm, src_hbm, dst_hbm, ew_hbm, out_hbm,
             srcv, dstv, ewv, rows, wbuf, zbuf, dtable, acc, sem_g, sem_s):
        cid = lax.axis_index("c")
        sid = lax.axis_index("s")
        wid = sid * _NC + cid
        z16 = jnp.zeros((16,), _F32)
        zi16 = jnp.zeros((16,), _I32)

        pltpu.sync_copy(dinv_hbm, dtable)

        @pl.loop(0, zr)
        def _(r2):
            for k in range(nkv):
                zbuf[r2, pl.ds(k * 16, 16)] = z16

        @pl.loop(sid, nzb, step=_NS)
        def _(zb):
            pltpu.sync_copy(zbuf, acc.at[pl.ds(zb * zr, zr)])

        plsc.subcore_barrier()

        @pl.loop(0, nb)
        def _(b):
            pltpu.sync_copy(src_hbm.at[wid, b], srcv)
            pltpu.sync_copy(dst_hbm.at[wid, b], dstv)
            pltpu.sync_copy(ew_hbm.at[wid, b], ewv)

            @pl.loop(0, _CB)
            def _(j):
                pltpu.async_copy(g_hbm.at[srcv.at[j]],
                                 rows.at[pl.ds(j * _R, _R)], sem_g)

            @pl.loop(0, _CB)
            def _(j):
                pltpu.make_async_copy(g_hbm.at[srcv.at[j]],
                                      rows.at[pl.ds(j * _R, _R)], sem_g).wait()

                for k in range(0, _R, 16):
                    sl = pl.ds(k, 16)
                    ws = (plsc.load_gather(dtable, [zi16, srcv[j, sl]])
                          * plsc.load_gather(dtable, [zi16, dstv[j, sl]])
                          * ewv[j, sl])
                    wbuf[sl] = ws

                rbase = j * _R

                @plsc.parallel_loop(0, _R, unroll=4)
                def _(e):
                    ws = plsc.load_gather(wbuf, [jnp.full((16,), e, _I32)])
                    row = rbase + e
                    for k in range(nkv):
                        sl = pl.ds(k * 16, 16)
                        rows[row, sl] = rows[row, sl] * ws

                pltpu.async_copy(rows.at[pl.ds(j * _R, _R)],
                                 acc.at[dstv.at[j]], sem_s, add=True)

            @pl.loop(0, _CB)
            def _(j):
                pltpu.make_async_copy(rows.at[pl.ds(j * _R, _R)],
                                      acc.at[dstv.at[j]], sem_s).wait()

        plsc.subcore_barrier()

        @pl.when(sid == 0)
        def _():
            pltpu.sync_copy(acc, out_hbm.at[cid])

    return pl.kernel(
        body,
        out_type=jax.ShapeDtypeStruct((_NC, n_nodes, dfeat), _F32),
        mesh=_sc_mesh(),
        compiler_params=_sc_params(),
        scratch_types=[
            pltpu.VMEM((_CB, _R), _I32),
            pltpu.VMEM((_CB, _R), _I32),
            pltpu.VMEM((_CB, _R), _F32),
            pltpu.VMEM((_CB * _R, dfeat), _F32),
            pltpu.VMEM((_R,), _F32),
            pltpu.VMEM((zr, dfeat), _F32),
            pltpu.VMEM((1, n_nodes), _F32),
            pltpu.VMEM_SHARED((n_nodes, dfeat), _F32),
            pltpu.SemaphoreType.DMA,
            pltpu.SemaphoreType.DMA,
        ],
    )


def _dot(a, b):
    return jax.lax.dot_general(a, b, (((1,), (0,)), ((), ())),
                               precision=jax.lax.Precision.HIGHEST,
                               preferred_element_type=_F32)


def _h1_body(x_ref, w1_ref, h1_ref):
    h1_ref[...] = _dot(x_ref[...], w1_ref[...])


def _mid_body(p1_ref, h1_ref, dinv_ref, b1_ref, w2_ref, h_ref, h2_ref):
    dinv = dinv_ref[...]
    x1 = (p1_ref[0] + p1_ref[1]) + (dinv * dinv) * h1_ref[...] + b1_ref[...]
    h = jnp.maximum(x1, 0.0)
    h_ref[...] = h
    h2_ref[...] = _dot(h, w2_ref[...])


def _final_body(p2_ref, h2_ref, h_ref, dinv_ref, b2_ref,
                fc1_wt_ref, fc1_b_ref, w_h_ref, w_c_ref, fc2_b_ref,
                r_ref, x1_out_ref):
    dinv = dinv_ref[...]
    x2 = (p2_ref[0] + p2_ref[1]) + (dinv * dinv) * h2_ref[...] + b2_ref[...]
    x1_out = _dot(x2, fc1_wt_ref[...]) + fc1_b_ref[...]
    x1_out_ref[...] = x1_out
    c = jax.nn.sigmoid(x1_out)
    r_ref[...] = _dot(h_ref[...], w_h_ref[...]) + c * w_c_ref[...] + fc2_b_ref[...]


_BN = 2000


def _row_spec(shape3=None, shape2=None):
    if shape3 is not None:
        return pl.BlockSpec((shape3[0], _BN, shape3[2]), lambda i: (0, i, 0))
    return pl.BlockSpec((_BN, shape2[1]), lambda i: (i, 0))


def _full_spec(shape):
    return pl.BlockSpec(shape, lambda i: tuple(0 for _ in shape))


def _tc_call(body, out_shapes):
    return pl.pallas_call(
        body,
        out_shape=[jax.ShapeDtypeStruct(s, _F32) for s in out_shapes])


def _tc_call_rows(body, n, out_minor, in_specs):
    grid = n // _BN
    return pl.pallas_call(
        body,
        grid=(grid,),
        in_specs=in_specs,
        out_specs=[_row_spec(shape2=(n, m)) for m in out_minor],
        out_shape=[jax.ShapeDtypeStruct((n, m), _F32) for m in out_minor])


def kernel(x, edge_index, edge_weights, W1, b1, W2, b2, fc1_W, fc1_b, fc2_W, fc2_b):
    n, d_in = x.shape
    e = edge_index.shape[1]
    d1 = W1.shape[1]
    d2 = W2.shape[1]
    epw = e // _NW
    nb = epw // (_CB * _R)

    eshape = (_NW, nb, _CB, _R)
    src4d = edge_index[0].reshape(eshape)
    dst4d = edge_index[1].reshape(eshape)
    ew4d = edge_weights.reshape(eshape)
    dst3d = dst4d.reshape(_NW, nb * _CB, _R)
    ew3d = ew4d.reshape(_NW, nb * _CB, _R)

    dinv_row = _deg_dinv_kernel(n, nb)(dst3d, ew3d)
    h1 = _tc_call(_h1_body, [(n, d1)])(x, W1)[0]
    dinv = dinv_row.reshape(n, 1)

    p1 = _agg_kernel(n, nb, d1)(h1, dinv_row, src4d, dst4d, ew4d)

    h, h2 = _tc_call_rows(
        _mid_body, n, [d1, d2],
        [_row_spec(shape3=(_NC, n, d1)), _row_spec(shape2=(n, d1)),
         _row_spec(shape2=(n, 1)), _full_spec((1, d1)), _full_spec((d1, d2))])(
        p1, h1, dinv, b1.reshape(1, d1), W2)

    p2 = _agg_kernel(n, nb, d2)(h2, dinv_row, src4d, dst4d, ew4d)

    r, x1_out = _tc_call_rows(
        _final_body, n, [1, 1],
        [_row_spec(shape3=(_NC, n, d2)), _row_spec(shape2=(n, d2)),
         _row_spec(shape2=(n, d1)), _row_spec(shape2=(n, 1)),
         _full_spec((1, d2)),
         _full_spec((d2, 1)), _full_spec((1, 1)),
         _full_spec((d1, 1)), _full_spec((1, 1)), _full_spec((1, 1))])(
        p2, h2, h, dinv, b2.reshape(1, d2),
        fc1_W.reshape(d2, 1), fc1_b.reshape(1, 1),
        fc2_W[:, :d1].reshape(d1, 1), fc2_W[:, d1:], fc2_b.reshape(1, 1))

    return (r, x1_out)


# speedup vs baseline: 49.8183x; 1.0686x over previous
import dataclasses
import functools

import jax
import jax.numpy as jnp
from jax import lax
from jax.experimental import pallas as pl
from jax.experimental.pallas import tpu as pltpu
from jax.experimental.pallas import tpu_sc as plsc

_F32 = jnp.float32
_I32 = jnp.int32
_NC = 2
_NS = 16
_NW = _NC * _NS
_R = 80
_CB = 25


def _sc_mesh():
    return plsc.VectorSubcoreMesh(core_axis_name="c", subcore_axis_name="s")


def _sc_params():
    cp = pltpu.CompilerParams()
    fields = pltpu.CompilerParams.__dataclass_fields__
    if "needs_layout_passes" in fields:
        cp = dataclasses.replace(cp, needs_layout_passes=False)
    if "use_tc_tiling_on_sc" in fields:
        cp = dataclasses.replace(cp, use_tc_tiling_on_sc=False)
    return cp


def _rsqrt_newton(d):
    i = plsc.bitcast(d, _I32)
    i = jnp.full((16,), 0x5F3759DF, _I32) - lax.shift_right_logical(
        i, jnp.full((16,), 1, _I32))
    y = plsc.bitcast(i, _F32)
    half = d * (-0.5)
    for _ in range(4):
        y = y * (half * y * y + 1.5)
    return y


@functools.cache
def _deg_dinv_kernel(n_nodes, nb):
    npart = nb * _CB
    nblk = n_nodes // _R

    def body(dst_hbm, ew_hbm, out_hbm, dstv, ewv, degloc, rbuf, dbuf,
             shdeg, sem):
        cid = lax.axis_index("c")
        sid = lax.axis_index("s")
        wid = sid * _NC + cid
        z16 = jnp.zeros((16,), _F32)
        zi16 = jnp.zeros((16,), _I32)

        @pl.loop(0, n_nodes, step=16)
        def _(i):
            degloc[0, pl.ds(i, 16)] = z16

        for half in range(2):
            p = sid * 2 + half
            cp_d = pltpu.async_copy(dst_hbm.at[p], dstv.at[half], sem)
            cp_w = pltpu.async_copy(ew_hbm.at[p], ewv.at[half], sem)
            cp_d.wait()
            cp_w.wait()

            @pl.loop(0, npart)
            def _(j):
                @pl.loop(0, _R, step=16)
                def _(k):
                    idx = dstv[half, j, pl.ds(k, 16)]
                    vals = ewv[half, j, pl.ds(k, 16)]
                    plsc.addupdate_scatter(degloc, [zi16, idx], vals)

        pltpu.sync_copy(degloc, shdeg.at[sid])
        plsc.subcore_barrier()

        @pl.loop(wid, nblk, step=_NW)
        def _(blk):
            pltpu.sync_copy(shdeg.at[:, 0, pl.ds(blk * _R, _R)], rbuf)

            @pl.loop(0, _R, step=16)
            def _(k):
                acc = rbuf[0, pl.ds(k, 16)] + 1.0
                for t in range(1, _NS):
                    acc = acc + rbuf[t, pl.ds(k, 16)]
                dbuf[0, pl.ds(k, 16)] = _rsqrt_newton(acc)

            pltpu.sync_copy(dbuf, out_hbm.at[:, pl.ds(blk * _R, _R)])

    return pl.kernel(
        body,
        out_type=jax.ShapeDtypeStruct((1, n_nodes), _F32),
        mesh=_sc_mesh(),
        compiler_params=_sc_params(),
        scratch_types=[
            pltpu.VMEM((2, npart, _R), _I32),
            pltpu.VMEM((2, npart, _R), _F32),
            pltpu.VMEM((1, n_nodes), _F32),
            pltpu.VMEM((_NS, _R), _F32),
            pltpu.VMEM((1, _R), _F32),
            pltpu.VMEM_SHARED((_NS, 1, n_nodes), _F32),
            pltpu.SemaphoreType.DMA,
        ],
    )


@functools.cache
def _agg_kernel(n_nodes, nb, dfeat):
    zr = 80
    nzb = n_nodes // zr
    nkv = dfeat // 16

    def body(g_hbm, dinv_hbm, src_hbm, dst_hbm, ew_hbm, out_hbm,
             srcv, dstv, ewv, rows, wbuf, zbuf, dtable, acc,
             sem_g, sem_s, sem_i):
        cid = lax.axis_index("c")
        sid = lax.axis_index("s")
        wid = sid * _NC + cid
        z16 = jnp.zeros((16,), _F32)
        zi16 = jnp.zeros((16,), _I32)

        pltpu.sync_copy(dinv_hbm, dtable)

        @pl.loop(0, zr)
        def _(r2):
            for k in range(nkv):
                zbuf[r2, pl.ds(k * 16, 16)] = z16

        @pl.loop(sid, nzb, step=_NS)
        def _(zb):
            pltpu.sync_copy(zbuf, acc.at[pl.ds(zb * zr, zr)])

        plsc.subcore_barrier()

        def fire_idx(b):
            par = b % 2
            sem = sem_i.at[par]
            pltpu.async_copy(src_hbm.at[wid, b], srcv.at[par], sem)
            pltpu.async_copy(dst_hbm.at[wid, b], dstv.at[par], sem)
            pltpu.async_copy(ew_hbm.at[wid, b], ewv.at[par], sem)

        def wait_idx(b):
            par = b % 2
            sem = sem_i.at[par]
            pltpu.make_async_copy(src_hbm.at[wid, b], srcv.at[par], sem).wait()
            pltpu.make_async_copy(dst_hbm.at[wid, b], dstv.at[par], sem).wait()
            pltpu.make_async_copy(ew_hbm.at[wid, b], ewv.at[par], sem).wait()

        fire_idx(0)
        for b in range(nb):
            par = b % 2
            wait_idx(b)
            if b + 1 < nb:
                fire_idx(b + 1)
            sv, dv, wv = srcv.at[par], dstv.at[par], ewv.at[par]

            @pl.loop(0, _CB)
            def _(j):
                pltpu.async_copy(g_hbm.at[sv.at[j]],
                                 rows.at[pl.ds(j * _R, _R)], sem_g)

            @pl.loop(0, _CB)
            def _(j):
                pltpu.make_async_copy(g_hbm.at[sv.at[j]],
                                      rows.at[pl.ds(j * _R, _R)], sem_g).wait()

                for k in range(0, _R, 16):
                    sl = pl.ds(k, 16)
                    ws = (plsc.load_gather(dtable, [zi16, sv[j, sl]])
                          * plsc.load_gather(dtable, [zi16, dv[j, sl]])
                          * wv[j, sl])
                    wbuf[sl] = ws

                rbase = j * _R

                @plsc.parallel_loop(0, _R, unroll=8)
                def _(e):
                    ws = plsc.load_gather(wbuf, [jnp.full((16,), e, _I32)])
                    row = rbase + e
                    for k in range(nkv):
                        sl = pl.ds(k * 16, 16)
                        rows[row, sl] = rows[row, sl] * ws

                pltpu.async_copy(rows.at[pl.ds(j * _R, _R)],
                                 acc.at[dv.at[j]], sem_s, add=True)

            @pl.loop(0, _CB)
            def _(j):
                pltpu.make_async_copy(rows.at[pl.ds(j * _R, _R)],
                                      acc.at[dv.at[j]], sem_s).wait()

        plsc.subcore_barrier()

        @pl.when(sid == 0)
        def _():
            pltpu.sync_copy(acc, out_hbm.at[cid])

    return pl.kernel(
        body,
        out_type=jax.ShapeDtypeStruct((_NC, n_nodes, dfeat), _F32),
        mesh=_sc_mesh(),
        compiler_params=_sc_params(),
        scratch_types=[
            pltpu.VMEM((2, _CB, _R), _I32),
            pltpu.VMEM((2, _CB, _R), _I32),
            pltpu.VMEM((2, _CB, _R), _F32),
            pltpu.VMEM((_CB * _R, dfeat), _F32),
            pltpu.VMEM((_R,), _F32),
            pltpu.VMEM((zr, dfeat), _F32),
            pltpu.VMEM((1, n_nodes), _F32),
            pltpu.VMEM_SHARED((n_nodes, dfeat), _F32),
            pltpu.SemaphoreType.DMA,
            pltpu.SemaphoreType.DMA,
            pltpu.SemaphoreType.DMA((2,)),
        ],
    )


def _dot(a, b):
    return jax.lax.dot_general(a, b, (((1,), (0,)), ((), ())),
                               precision=jax.lax.Precision.HIGHEST,
                               preferred_element_type=_F32)


def _h1_body(x_ref, w1_ref, h1_ref):
    h1_ref[...] = _dot(x_ref[...], w1_ref[...])


def _mid_body(p1_ref, h1_ref, dinv_ref, b1_ref, w2_ref, h_ref, h2_ref):
    dinv = dinv_ref[...]
    x1 = (p1_ref[0] + p1_ref[1]) + (dinv * dinv) * h1_ref[...] + b1_ref[...]
    h = jnp.maximum(x1, 0.0)
    h_ref[...] = h
    h2_ref[...] = _dot(h, w2_ref[...])


def _final_body(p2_ref, h2_ref, h_ref, dinv_ref, b2_ref,
                fc1_wt_ref, fc1_b_ref, w_h_ref, w_c_ref, fc2_b_ref,
                r_ref, x1_out_ref):
    dinv = dinv_ref[...]
    x2 = (p2_ref[0] + p2_ref[1]) + (dinv * dinv) * h2_ref[...] + b2_ref[...]
    x1_out = _dot(x2, fc1_wt_ref[...]) + fc1_b_ref[...]
    x1_out_ref[...] = x1_out
    c = jax.nn.sigmoid(x1_out)
    r_ref[...] = _dot(h_ref[...], w_h_ref[...]) + c * w_c_ref[...] + fc2_b_ref[...]


_BN = 2000


def _row_spec(shape3=None, shape2=None):
    if shape3 is not None:
        return pl.BlockSpec((shape3[0], _BN, shape3[2]), lambda i: (0, i, 0))
    return pl.BlockSpec((_BN, shape2[1]), lambda i: (i, 0))


def _full_spec(shape):
    return pl.BlockSpec(shape, lambda i: tuple(0 for _ in shape))


def _tc_call(body, out_shapes):
    return pl.pallas_call(
        body,
        out_shape=[jax.ShapeDtypeStruct(s, _F32) for s in out_shapes])


def _tc_call_rows(body, n, out_minor, in_specs):
    grid = n // _BN
    return pl.pallas_call(
        body,
        grid=(grid,),
        in_specs=in_specs,
        out_specs=[_row_spec(shape2=(n, m)) for m in out_minor],
        out_shape=[jax.ShapeDtypeStruct((n, m), _F32) for m in out_minor])


def kernel(x, edge_index, edge_weights, W1, b1, W2, b2, fc1_W, fc1_b, fc2_W, fc2_b):
    n, d_in = x.shape
    e = edge_index.shape[1]
    d1 = W1.shape[1]
    d2 = W2.shape[1]
    epw = e // _NW
    nb = epw // (_CB * _R)

    eshape = (_NW, nb, _CB, _R)
    src4d = edge_index[0].reshape(eshape)
    dst4d = edge_index[1].reshape(eshape)
    ew4d = edge_weights.reshape(eshape)
    dst3d = dst4d.reshape(_NW, nb * _CB, _R)
    ew3d = ew4d.reshape(_NW, nb * _CB, _R)

    dinv_row = _deg_dinv_kernel(n, nb)(dst3d, ew3d)
    h1 = _tc_call(_h1_body, [(n, d1)])(x, W1)[0]
    dinv = dinv_row.reshape(n, 1)

    p1 = _agg_kernel(n, nb, d1)(h1, dinv_row, src4d, dst4d, ew4d)

    h, h2 = _tc_call_rows(
        _mid_body, n, [d1, d2],
        [_row_spec(shape3=(_NC, n, d1)), _row_spec(shape2=(n, d1)),
         _row_spec(shape2=(n, 1)), _full_spec((1, d1)), _full_spec((d1, d2))])(
        p1, h1, dinv, b1.reshape(1, d1), W2)

    p2 = _agg_kernel(n, nb, d2)(h2, dinv_row, src4d, dst4d, ew4d)

    r, x1_out = _tc_call_rows(
        _final_body, n, [1, 1],
        [_row_spec(shape3=(_NC, n, d2)), _row_spec(shape2=(n, d2)),
         _row_spec(shape2=(n, d1)), _row_spec(shape2=(n, 1)),
         _full_spec((1, d2)),
         _full_spec((d2, 1)), _full_spec((1, 1)),
         _full_spec((d1, 1)), _full_spec((1, 1)), _full_spec((1, 1))])(
        p2, h2, h, dinv, b2.reshape(1, d2),
        fc1_W.reshape(d2, 1), fc1_b.reshape(1, 1),
        fc2_W[:, :d1].reshape(d1, 1), fc2_W[:, d1:], fc2_b.reshape(1, 1))

    return (r, x1_out)


# speedup vs baseline: 49.9823x; 1.0033x over previous
import dataclasses
import functools

import jax
import jax.numpy as jnp
from jax import lax
from jax.experimental import pallas as pl
from jax.experimental.pallas import tpu as pltpu
from jax.experimental.pallas import tpu_sc as plsc

_F32 = jnp.float32
_I32 = jnp.int32
_NC = 2
_NS = 16
_NW = _NC * _NS
_R = 80
_CB = 25


def _sc_mesh():
    return plsc.VectorSubcoreMesh(core_axis_name="c", subcore_axis_name="s")


def _sc_params():
    cp = pltpu.CompilerParams()
    fields = pltpu.CompilerParams.__dataclass_fields__
    if "needs_layout_passes" in fields:
        cp = dataclasses.replace(cp, needs_layout_passes=False)
    if "use_tc_tiling_on_sc" in fields:
        cp = dataclasses.replace(cp, use_tc_tiling_on_sc=False)
    return cp


def _rsqrt_newton(d):
    i = plsc.bitcast(d, _I32)
    i = jnp.full((16,), 0x5F3759DF, _I32) - lax.shift_right_logical(
        i, jnp.full((16,), 1, _I32))
    y = plsc.bitcast(i, _F32)
    half = d * (-0.5)
    for _ in range(4):
        y = y * (half * y * y + 1.5)
    return y


@functools.cache
def _agg_kernel(n_nodes, nb, dfeat, compute_dinv):
    zr = 80
    nzb = n_nodes // zr
    nkv = dfeat // 16
    rc = 80
    nrc = n_nodes // rc

    def body(g_hbm, src_hbm, dst_hbm, ew_hbm, *refs):
        if compute_dinv:
            (out_hbm, dinv_out,
             srcv, dstv, ewv, rows, wbuf, zbuf, dtable,
             rbuf, dbuf, acc, shdeg, shdinv,
             sem_g, sem_s, sem_i) = refs
            degloc = dtable
        else:
            (dinv_hbm, out_hbm,
             srcv, dstv, ewv, rows, wbuf, zbuf, dtable, acc,
             sem_g, sem_s, sem_i) = refs
        cid = lax.axis_index("c")
        sid = lax.axis_index("s")
        wid = sid * _NC + cid
        z16 = jnp.zeros((16,), _F32)
        zi16 = jnp.zeros((16,), _I32)

        @pl.loop(0, zr)
        def _(r2):
            for k in range(nkv):
                zbuf[r2, pl.ds(k * 16, 16)] = z16

        @pl.loop(sid, nzb, step=_NS)
        def _(zb):
            pltpu.sync_copy(zbuf, acc.at[pl.ds(zb * zr, zr)])

        if compute_dinv:
            @pl.loop(0, n_nodes, step=16)
            def _(i):
                degloc[0, pl.ds(i, 16)] = z16

            def fire_de(t):
                par = t % 2
                sem = sem_i.at[par]
                p = sid * 2 + t // nb
                b = t % nb
                pltpu.async_copy(dst_hbm.at[p, b], dstv.at[par], sem)
                pltpu.async_copy(ew_hbm.at[p, b], ewv.at[par], sem)

            def wait_de(t):
                par = t % 2
                sem = sem_i.at[par]
                p = sid * 2 + t // nb
                b = t % nb
                pltpu.make_async_copy(dst_hbm.at[p, b], dstv.at[par], sem).wait()
                pltpu.make_async_copy(ew_hbm.at[p, b], ewv.at[par], sem).wait()

            fire_de(0)
            for t in range(2 * nb):
                wait_de(t)
                if t + 1 < 2 * nb:
                    fire_de(t + 1)
                par = t % 2

                @pl.loop(0, _CB)
                def _(j):
                    @pl.loop(0, _R, step=16)
                    def _(k):
                        sl = pl.ds(k, 16)
                        plsc.addupdate_scatter(
                            degloc, [zi16, dstv[par, j, sl]], ewv[par, j, sl])

            pltpu.sync_copy(degloc, shdeg.at[sid])
            plsc.subcore_barrier()

            @pl.loop(sid, nrc, step=_NS)
            def _(ci):
                pltpu.sync_copy(shdeg.at[:, 0, pl.ds(ci * rc, rc)], rbuf)

                @pl.loop(0, rc, step=16)
                def _(k):
                    sl = pl.ds(k, 16)
                    a16 = rbuf[0, sl] + 1.0
                    for t in range(1, _NS):
                        a16 = a16 + rbuf[t, sl]
                    dbuf[0, sl] = _rsqrt_newton(a16)

                pltpu.sync_copy(dbuf, shdinv.at[:, pl.ds(ci * rc, rc)])

                @pl.when(cid == 0)
                def _():
                    pltpu.sync_copy(dbuf, dinv_out.at[:, pl.ds(ci * rc, rc)])

            plsc.subcore_barrier()
            pltpu.sync_copy(shdinv, dtable)
        else:
            pltpu.sync_copy(dinv_hbm, dtable)
            plsc.subcore_barrier()

        def fire_idx(b):
            par = b % 2
            sem = sem_i.at[par]
            pltpu.async_copy(src_hbm.at[wid, b], srcv.at[par], sem)
            pltpu.async_copy(dst_hbm.at[wid, b], dstv.at[par], sem)
            pltpu.async_copy(ew_hbm.at[wid, b], ewv.at[par], sem)

        def wait_idx(b):
            par = b % 2
            sem = sem_i.at[par]
            pltpu.make_async_copy(src_hbm.at[wid, b], srcv.at[par], sem).wait()
            pltpu.make_async_copy(dst_hbm.at[wid, b], dstv.at[par], sem).wait()
            pltpu.make_async_copy(ew_hbm.at[wid, b], ewv.at[par], sem).wait()

        fire_idx(0)
        for b in range(nb):
            par = b % 2
            wait_idx(b)
            if b + 1 < nb:
                fire_idx(b + 1)
            sv, dv, wv = srcv.at[par], dstv.at[par], ewv.at[par]

            @pl.loop(0, _CB)
            def _(j):
                pltpu.async_copy(g_hbm.at[sv.at[j]],
                                 rows.at[pl.ds(j * _R, _R)], sem_g)

            @pl.loop(0, _CB)
            def _(j):
                pltpu.make_async_copy(g_hbm.at[sv.at[j]],
                                      rows.at[pl.ds(j * _R, _R)], sem_g).wait()

                for k in range(0, _R, 16):
                    sl = pl.ds(k, 16)
                    ws = (plsc.load_gather(dtable, [zi16, sv[j, sl]])
                          * plsc.load_gather(dtable, [zi16, dv[j, sl]])
                          * wv[j, sl])
                    wbuf[sl] = ws

                rbase = j * _R

                @plsc.parallel_loop(0, _R, unroll=8)
                def _(e):
                    ws = plsc.load_gather(wbuf, [jnp.full((16,), e, _I32)])
                    row = rbase + e
                    for k in range(nkv):
                        sl = pl.ds(k * 16, 16)
                        rows[row, sl] = rows[row, sl] * ws

                pltpu.async_copy(rows.at[pl.ds(j * _R, _R)],
                                 acc.at[dv.at[j]], sem_s, add=True)

            @pl.loop(0, _CB)
            def _(j):
                pltpu.make_async_copy(rows.at[pl.ds(j * _R, _R)],
                                      acc.at[dv.at[j]], sem_s).wait()

        plsc.subcore_barrier()

        @pl.when(sid == 0)
        def _():
            pltpu.sync_copy(acc, out_hbm.at[cid])

    out_types = [jax.ShapeDtypeStruct((_NC, n_nodes, dfeat), _F32)]
    scratch = [
        pltpu.VMEM((2, _CB, _R), _I32),
        pltpu.VMEM((2, _CB, _R), _I32),
        pltpu.VMEM((2, _CB, _R), _F32),
        pltpu.VMEM((_CB * _R, dfeat), _F32),
        pltpu.VMEM((_R,), _F32),
        pltpu.VMEM((zr, dfeat), _F32),
        pltpu.VMEM((1, n_nodes), _F32),
    ]
    if compute_dinv:
        out_types.append(jax.ShapeDtypeStruct((1, n_nodes), _F32))
        scratch += [
            pltpu.VMEM((_NS, rc), _F32),
            pltpu.VMEM((1, rc), _F32),
            pltpu.VMEM_SHARED((n_nodes, dfeat), _F32),
            pltpu.VMEM_SHARED((_NS, 1, n_nodes), _F32),
            pltpu.VMEM_SHARED((1, n_nodes), _F32),
        ]
    else:
        scratch += [
            pltpu.VMEM_SHARED((n_nodes, dfeat), _F32),
        ]
    scratch += [
        pltpu.SemaphoreType.DMA,
        pltpu.SemaphoreType.DMA,
        pltpu.SemaphoreType.DMA((2,)),
    ]

    return pl.kernel(
        body,
        out_type=out_types,
        mesh=_sc_mesh(),
        compiler_params=_sc_params(),
        scratch_types=scratch,
    )


def _dot(a, b):
    return jax.lax.dot_general(a, b, (((1,), (0,)), ((), ())),
                               preferred_element_type=_F32)


def _h1_body(x_ref, w1_ref, h1_ref):
    h1_ref[...] = _dot(x_ref[...], w1_ref[...])


def _mid_body(p1_ref, h1_ref, dinv_ref, b1_ref, w2_ref, h_ref, h2_ref):
    dinv = dinv_ref[...]
    x1 = (p1_ref[0] + p1_ref[1]) + (dinv * dinv) * h1_ref[...] + b1_ref[...]
    h = jnp.maximum(x1, 0.0)
    h_ref[...] = h
    h2_ref[...] = _dot(h, w2_ref[...])


def _final_body(p2_ref, h2_ref, h_ref, dinv_ref, b2_ref,
                fc1_wt_ref, fc1_b_ref, w_h_ref, w_c_ref, fc2_b_ref,
                r_ref, x1_out_ref):
    dinv = dinv_ref[...]
    x2 = (p2_ref[0] + p2_ref[1]) + (dinv * dinv) * h2_ref[...] + b2_ref[...]
    x1_out = _dot(x2, fc1_wt_ref[...]) + fc1_b_ref[...]
    x1_out_ref[...] = x1_out
    c = jax.nn.sigmoid(x1_out)
    r_ref[...] = _dot(h_ref[...], w_h_ref[...]) + c * w_c_ref[...] + fc2_b_ref[...]


_BN = 2000


def _row_spec(shape3=None, shape2=None):
    if shape3 is not None:
        return pl.BlockSpec((shape3[0], _BN, shape3[2]), lambda i: (0, i, 0))
    return pl.BlockSpec((_BN, shape2[1]), lambda i: (i, 0))


def _full_spec(shape):
    return pl.BlockSpec(shape, lambda i: tuple(0 for _ in shape))


def _tc_call(body, out_shapes):
    return pl.pallas_call(
        body,
        out_shape=[jax.ShapeDtypeStruct(s, _F32) for s in out_shapes])


def _tc_call_rows(body, n, out_minor, in_specs):
    grid = n // _BN
    return pl.pallas_call(
        body,
        grid=(grid,),
        in_specs=in_specs,
        out_specs=[_row_spec(shape2=(n, m)) for m in out_minor],
        out_shape=[jax.ShapeDtypeStruct((n, m), _F32) for m in out_minor])


def kernel(x, edge_index, edge_weights, W1, b1, W2, b2, fc1_W, fc1_b, fc2_W, fc2_b):
    n, d_in = x.shape
    e = edge_index.shape[1]
    d1 = W1.shape[1]
    d2 = W2.shape[1]
    epw = e // _NW
    nb = epw // (_CB * _R)

    eshape = (_NW, nb, _CB, _R)
    src4d = edge_index[0].reshape(eshape)
    dst4d = edge_index[1].reshape(eshape)
    ew4d = edge_weights.reshape(eshape)

    h1 = _tc_call(_h1_body, [(n, d1)])(x, W1)[0]

    p1, dinv_row = _agg_kernel(n, nb, d1, True)(h1, src4d, dst4d, ew4d)
    dinv = dinv_row.reshape(n, 1)

    h, h2 = _tc_call_rows(
        _mid_body, n, [d1, d2],
        [_row_spec(shape3=(_NC, n, d1)), _row_spec(shape2=(n, d1)),
         _row_spec(shape2=(n, 1)), _full_spec((1, d1)), _full_spec((d1, d2))])(
        p1, h1, dinv, b1.reshape(1, d1), W2)

    p2, = _agg_kernel(n, nb, d2, False)(h2, src4d, dst4d, ew4d, dinv_row)

    r, x1_out = _tc_call_rows(
        _final_body, n, [1, 1],
        [_row_spec(shape3=(_NC, n, d2)), _row_spec(shape2=(n, d2)),
         _row_spec(shape2=(n, d1)), _row_spec(shape2=(n, 1)),
         _full_spec((1, d2)),
         _full_spec((d2, 1)), _full_spec((1, 1)),
         _full_spec((d1, 1)), _full_spec((1, 1)), _full_spec((1, 1))])(
        p2, h2, h, dinv, b2.reshape(1, d2),
        fc1_W.reshape(d2, 1), fc1_b.reshape(1, 1),
        fc2_W[:, :d1].reshape(d1, 1), fc2_W[:, d1:], fc2_b.reshape(1, 1))

    return (r, x1_out)


# speedup vs baseline: 50.3085x; 1.0065x over previous
import dataclasses
import functools

import jax
import jax.numpy as jnp
from jax import lax
from jax.experimental import pallas as pl
from jax.experimental.pallas import tpu as pltpu
from jax.experimental.pallas import tpu_sc as plsc

_F32 = jnp.float32
_I32 = jnp.int32
_NC = 2
_NS = 16
_NW = _NC * _NS
_R = 80
_CB = 25


def _sc_mesh():
    return plsc.VectorSubcoreMesh(core_axis_name="c", subcore_axis_name="s")


def _sc_params():
    cp = pltpu.CompilerParams()
    fields = pltpu.CompilerParams.__dataclass_fields__
    if "needs_layout_passes" in fields:
        cp = dataclasses.replace(cp, needs_layout_passes=False)
    if "use_tc_tiling_on_sc" in fields:
        cp = dataclasses.replace(cp, use_tc_tiling_on_sc=False)
    return cp


def _rsqrt_newton(d):
    i = plsc.bitcast(d, _I32)
    i = jnp.full((16,), 0x5F3759DF, _I32) - lax.shift_right_logical(
        i, jnp.full((16,), 1, _I32))
    y = plsc.bitcast(i, _F32)
    half = d * (-0.5)
    for _ in range(4):
        y = y * (half * y * y + 1.5)
    return y


@functools.cache
def _agg_kernel(n_nodes, nb, dfeat, compute_dinv):
    zr = 80
    nzb = n_nodes // zr
    nkv = dfeat // 16
    rc = 400
    nrc = n_nodes // rc

    def body(g_hbm, src_hbm, dst_hbm, ew_hbm, *refs):
        if compute_dinv:
            (out_hbm, dinv_out,
             srcv, dstv, ewv, rows, wbuf, zbuf, dtable,
             rbuf, dbuf, acc, shdeg, shdinv,
             sem_g, sem_s, sem_i) = refs
            degloc = dtable
        else:
            (dinv_hbm, out_hbm,
             srcv, dstv, ewv, rows, wbuf, zbuf, dtable, acc,
             sem_g, sem_s, sem_i) = refs
        cid = lax.axis_index("c")
        sid = lax.axis_index("s")
        wid = sid * _NC + cid
        z16 = jnp.zeros((16,), _F32)
        zi16 = jnp.zeros((16,), _I32)

        @pl.loop(0, zr)
        def _(r2):
            for k in range(nkv):
                zbuf[r2, pl.ds(k * 16, 16)] = z16

        @pl.loop(sid, nzb, step=_NS)
        def _(zb):
            pltpu.sync_copy(zbuf, acc.at[pl.ds(zb * zr, zr)])

        if compute_dinv:
            @pl.loop(0, n_nodes, step=16)
            def _(i):
                degloc[0, pl.ds(i, 16)] = z16

            def fire_de(t):
                par = t % 2
                sem = sem_i.at[par]
                p = sid * 2 + t // nb
                b = t % nb
                pltpu.async_copy(dst_hbm.at[p, b], dstv.at[par], sem)
                pltpu.async_copy(ew_hbm.at[p, b], ewv.at[par], sem)

            def wait_de(t):
                par = t % 2
                sem = sem_i.at[par]
                p = sid * 2 + t // nb
                b = t % nb
                pltpu.make_async_copy(dst_hbm.at[p, b], dstv.at[par], sem).wait()
                pltpu.make_async_copy(ew_hbm.at[p, b], ewv.at[par], sem).wait()

            fire_de(0)
            for t in range(2 * nb):
                wait_de(t)
                if t + 1 < 2 * nb:
                    fire_de(t + 1)
                par = t % 2

                @pl.loop(0, _CB)
                def _(j):
                    @pl.loop(0, _R, step=16)
                    def _(k):
                        sl = pl.ds(k, 16)
                        plsc.addupdate_scatter(
                            degloc, [zi16, dstv[par, j, sl]], ewv[par, j, sl])

            pltpu.sync_copy(degloc, shdeg.at[sid])
            plsc.subcore_barrier()

            @pl.loop(sid, nrc, step=_NS)
            def _(ci):
                pltpu.sync_copy(shdeg.at[:, 0, pl.ds(ci * rc, rc)], rbuf)

                @pl.loop(0, rc, step=16)
                def _(k):
                    sl = pl.ds(k, 16)
                    a16 = rbuf[0, sl] + 1.0
                    for t in range(1, _NS):
                        a16 = a16 + rbuf[t, sl]
                    dbuf[0, sl] = _rsqrt_newton(a16)

                pltpu.sync_copy(dbuf, shdinv.at[:, pl.ds(ci * rc, rc)])

                @pl.when(cid == 0)
                def _():
                    pltpu.sync_copy(dbuf, dinv_out.at[:, pl.ds(ci * rc, rc)])

            plsc.subcore_barrier()
            pltpu.sync_copy(shdinv, dtable)
        else:
            pltpu.sync_copy(dinv_hbm, dtable)
            plsc.subcore_barrier()

        def fire_idx(b):
            par = b % 2
            sem = sem_i.at[par]
            pltpu.async_copy(src_hbm.at[wid, b], srcv.at[par], sem)
            pltpu.async_copy(dst_hbm.at[wid, b], dstv.at[par], sem)
            pltpu.async_copy(ew_hbm.at[wid, b], ewv.at[par], sem)

        def wait_idx(b):
            par = b % 2
            sem = sem_i.at[par]
            pltpu.make_async_copy(src_hbm.at[wid, b], srcv.at[par], sem).wait()
            pltpu.make_async_copy(dst_hbm.at[wid, b], dstv.at[par], sem).wait()
            pltpu.make_async_copy(ew_hbm.at[wid, b], ewv.at[par], sem).wait()

        fire_idx(0)
        for b in range(nb):
            par = b % 2
            wait_idx(b)
            if b + 1 < nb:
                fire_idx(b + 1)
            sv, dv, wv = srcv.at[par], dstv.at[par], ewv.at[par]

            @pl.loop(0, _CB)
            def _(j):
                pltpu.async_copy(g_hbm.at[sv.at[j]],
                                 rows.at[pl.ds(j * _R, _R)], sem_g)

            @pl.loop(0, _CB)
            def _(j):
                pltpu.make_async_copy(g_hbm.at[sv.at[j]],
                                      rows.at[pl.ds(j * _R, _R)], sem_g).wait()

                for k in range(0, _R, 16):
                    sl = pl.ds(k, 16)
                    ws = (plsc.load_gather(dtable, [zi16, sv[j, sl]])
                          * plsc.load_gather(dtable, [zi16, dv[j, sl]])
                          * wv[j, sl])
                    wbuf[sl] = ws

                rbase = j * _R

                @plsc.parallel_loop(0, _R, unroll=8)
                def _(e):
                    ws = plsc.load_gather(wbuf, [jnp.full((16,), e, _I32)])
                    row = rbase + e
                    for k in range(nkv):
                        sl = pl.ds(k * 16, 16)
                        rows[row, sl] = rows[row, sl] * ws

                pltpu.async_copy(rows.at[pl.ds(j * _R, _R)],
                                 acc.at[dv.at[j]], sem_s, add=True)

            @pl.loop(0, _CB)
            def _(j):
                pltpu.make_async_copy(rows.at[pl.ds(j * _R, _R)],
                                      acc.at[dv.at[j]], sem_s).wait()

        plsc.subcore_barrier()

        @pl.loop(sid, nzb, step=_NS)
        def _(zb):
            pltpu.async_copy(acc.at[pl.ds(zb * zr, zr)],
                             out_hbm.at[cid, pl.ds(zb * zr, zr)], sem_g)

        @pl.loop(sid, nzb, step=_NS)
        def _(zb):
            pltpu.make_async_copy(acc.at[pl.ds(zb * zr, zr)],
                                  out_hbm.at[cid, pl.ds(zb * zr, zr)], sem_g).wait()

    out_types = [jax.ShapeDtypeStruct((_NC, n_nodes, dfeat), _F32)]
    scratch = [
        pltpu.VMEM((2, _CB, _R), _I32),
        pltpu.VMEM((2, _CB, _R), _I32),
        pltpu.VMEM((2, _CB, _R), _F32),
        pltpu.VMEM((_CB * _R, dfeat), _F32),
        pltpu.VMEM((_R,), _F32),
        pltpu.VMEM((zr, dfeat), _F32),
        pltpu.VMEM((1, n_nodes), _F32),
    ]
    if compute_dinv:
        out_types.append(jax.ShapeDtypeStruct((1, n_nodes), _F32))
        scratch += [
            pltpu.VMEM((_NS, rc), _F32),
            pltpu.VMEM((1, rc), _F32),
            pltpu.VMEM_SHARED((n_nodes, dfeat), _F32),
            pltpu.VMEM_SHARED((_NS, 1, n_nodes), _F32),
            pltpu.VMEM_SHARED((1, n_nodes), _F32),
        ]
    else:
        scratch += [
            pltpu.VMEM_SHARED((n_nodes, dfeat), _F32),
        ]
    scratch += [
        pltpu.SemaphoreType.DMA,
        pltpu.SemaphoreType.DMA,
        pltpu.SemaphoreType.DMA((2,)),
    ]

    return pl.kernel(
        body,
        out_type=out_types,
        mesh=_sc_mesh(),
        compiler_params=_sc_params(),
        scratch_types=scratch,
    )


def _dot(a, b):
    return jax.lax.dot_general(a, b, (((1,), (0,)), ((), ())),
                               preferred_element_type=_F32)


def _h1_body(x_ref, w1_ref, h1_ref):
    h1_ref[...] = _dot(x_ref[...], w1_ref[...])


def _mid_body(p1_ref, h1_ref, dinv_ref, b1_ref, w2_ref, h_ref, h2_ref):
    dinv = dinv_ref[...]
    x1 = (p1_ref[0] + p1_ref[1]) + (dinv * dinv) * h1_ref[...] + b1_ref[...]
    h = jnp.maximum(x1, 0.0)
    h_ref[...] = h
    h2_ref[...] = _dot(h, w2_ref[...])


def _final_body(p2_ref, h2_ref, h_ref, dinv_ref, b2_ref,
                fc1_wt_ref, fc1_b_ref, w_h_ref, w_c_ref, fc2_b_ref,
                r_ref, x1_out_ref):
    dinv = dinv_ref[...]
    x2 = (p2_ref[0] + p2_ref[1]) + (dinv * dinv) * h2_ref[...] + b2_ref[...]
    x1_out = _dot(x2, fc1_wt_ref[...]) + fc1_b_ref[...]
    x1_out_ref[...] = x1_out
    c = jax.nn.sigmoid(x1_out)
    r_ref[...] = _dot(h_ref[...], w_h_ref[...]) + c * w_c_ref[...] + fc2_b_ref[...]


_BN = 2000


def _row_spec(shape3=None, shape2=None):
    if shape3 is not None:
        return pl.BlockSpec((shape3[0], _BN, shape3[2]), lambda i: (0, i, 0))
    return pl.BlockSpec((_BN, shape2[1]), lambda i: (i, 0))


def _full_spec(shape):
    return pl.BlockSpec(shape, lambda i: tuple(0 for _ in shape))


def _tc_call(body, out_shapes):
    return pl.pallas_call(
        body,
        out_shape=[jax.ShapeDtypeStruct(s, _F32) for s in out_shapes])


def _tc_call_rows(body, n, out_minor, in_specs):
    grid = n // _BN
    return pl.pallas_call(
        body,
        grid=(grid,),
        in_specs=in_specs,
        out_specs=[_row_spec(shape2=(n, m)) for m in out_minor],
        out_shape=[jax.ShapeDtypeStruct((n, m), _F32) for m in out_minor])


def kernel(x, edge_index, edge_weights, W1, b1, W2, b2, fc1_W, fc1_b, fc2_W, fc2_b):
    n, d_in = x.shape
    e = edge_index.shape[1]
    d1 = W1.shape[1]
    d2 = W2.shape[1]
    epw = e // _NW
    nb = epw // (_CB * _R)

    eshape = (_NW, nb, _CB, _R)
    src4d = edge_index[0].reshape(eshape)
    dst4d = edge_index[1].reshape(eshape)
    ew4d = edge_weights.reshape(eshape)

    h1 = _tc_call(_h1_body, [(n, d1)])(x, W1)[0]

    p1, dinv_row = _agg_kernel(n, nb, d1, True)(h1, src4d, dst4d, ew4d)
    dinv = dinv_row.reshape(n, 1)

    h, h2 = _tc_call_rows(
        _mid_body, n, [d1, d2],
        [_row_spec(shape3=(_NC, n, d1)), _row_spec(shape2=(n, d1)),
         _row_spec(shape2=(n, 1)), _full_spec((1, d1)), _full_spec((d1, d2))])(
        p1, h1, dinv, b1.reshape(1, d1), W2)

    p2, = _agg_kernel(n, nb, d2, False)(h2, src4d, dst4d, ew4d, dinv_row)

    r, x1_out = _tc_call_rows(
        _final_body, n, [1, 1],
        [_row_spec(shape3=(_NC, n, d2)), _row_spec(shape2=(n, d2)),
         _row_spec(shape2=(n, d1)), _row_spec(shape2=(n, 1)),
         _full_spec((1, d2)),
         _full_spec((d2, 1)), _full_spec((1, 1)),
         _full_spec((d1, 1)), _full_spec((1, 1)), _full_spec((1, 1))])(
        p2, h2, h, dinv, b2.reshape(1, d2),
        fc1_W.reshape(d2, 1), fc1_b.reshape(1, 1),
        fc2_W[:, :d1].reshape(d1, 1), fc2_W[:, d1:], fc2_b.reshape(1, 1))

    return (r, x1_out)
